# baseline (speedup 1.0000x reference)
import functools

import jax
import jax.numpy as jnp
from jax import lax
from jax.experimental import pallas as pl
from jax.experimental.pallas import tpu as pltpu
from jax.experimental.pallas import tpu_sc as plsc

_B = 16384
_L = 50
_EMB = 64
_TOTAL = _B * _L
_NC = 2
_NS = 16
_NW = _NC * _NS
_PER_W = _TOTAL // _NW
_CHUNK = 256
_NCH = _PER_W // _CHUNK
_NBUF = 4


@jax.jit
def _gather_call(table, idx3):
    mesh = plsc.VectorSubcoreMesh(core_axis_name="c", subcore_axis_name="s")

    @functools.partial(
        pl.kernel,
        mesh=mesh,
        out_type=jax.ShapeDtypeStruct((_TOTAL, _EMB), jnp.float32),
        scratch_types=(
            [pltpu.VMEM((_NCH, _CHUNK), jnp.int32)]
            + [pltpu.VMEM((_CHUNK, _EMB), jnp.float32)] * _NBUF
            + [pltpu.SemaphoreType.DMA] * (2 * _NBUF)
        ),
        compiler_params=pltpu.CompilerParams(use_tc_tiling_on_sc=False),
    )
    def k(table_hbm, idx_hbm, out_hbm, idx_v, *bufs):
        rows = bufs[:_NBUF]
        gsem = bufs[_NBUF:2 * _NBUF]
        osem = bufs[2 * _NBUF:]
        wid = lax.axis_index("s") * _NC + lax.axis_index("c")
        base = wid * _PER_W
        pltpu.sync_copy(idx_hbm.at[wid], idx_v)

        def start_gather(j, b):
            pltpu.async_copy(table_hbm.at[idx_v.at[j]], rows[b], gsem[b])

        def wait_gather(j, b):
            pltpu.make_async_copy(table_hbm.at[idx_v.at[j]], rows[b],
                                  gsem[b]).wait()

        def out_slice(j):
            return out_hbm.at[pl.ds(base + j * _CHUNK, _CHUNK)]

        def start_store(j, b):
            pltpu.async_copy(rows[b], out_slice(j), osem[b])

        def wait_store(j, b):
            pltpu.make_async_copy(rows[b], out_slice(j), osem[b]).wait()

        for j in range(_NBUF - 1):
            start_gather(j, j)

        def body(i, carry):
            for b in range(_NBUF):
                j = i * _NBUF + b
                prv = (b - 1) % _NBUF

                @pl.when(j + _NBUF - 1 < _NCH)
                def _():
                    @pl.when(j >= 1)
                    def _():
                        wait_store(j - 1, prv)
                    start_gather(j + _NBUF - 1, prv)

                wait_gather(j, b)
                start_store(j, b)
            return carry

        lax.fori_loop(0, _NCH // _NBUF, body, 0)
        for j in range(_NCH - _NBUF, _NCH):
            wait_store(j, j % _NBUF)

    return k(table, idx3)


def kernel(data, iword_indicator, iword_numerals, ivectors_weight,
           gmm_posterior, iprototypes_embeddings):
    idx3 = data.reshape(_NW, _NCH, _CHUNK)
    out = _gather_call(ivectors_weight, idx3)
    return out.reshape(_B, _L, _EMB)
